# baseline (speedup 1.0000x reference)
import functools

import jax
import jax.numpy as jnp
from jax import lax
from jax.experimental import pallas as pl
from jax.experimental.pallas import tpu as pltpu
from jax.experimental.pallas import tpu_sc as plsc

N = 10000
M = 10000
NNZ = 160000
IN_DIM = 256
HID = 256
OUT = 64
PROJ = 64
EPS = 1e-5

NC = 2
NS = 16
BLK = 125
NB1 = NNZ // NS // BLK
NB2 = NNZ // (NC * NS) // BLK
BLK1 = 50
NBP1 = NNZ // NS // BLK1
D = 128
NPAD = 10240
RPT = NPAD // NS


def _linbn_body(x_ref, w_ref, b_ref, g_ref, be_ref, o_ref):
    h = jnp.dot(x_ref[...], w_ref[...], preferred_element_type=jnp.float32)
    h = h + b_ref[...]
    mu = jnp.mean(h, axis=0, keepdims=True)
    var = jnp.mean((h - mu) ** 2, axis=0, keepdims=True)
    h = (h - mu) * lax.rsqrt(var + EPS) * g_ref[...] + be_ref[...]
    o_ref[0] = h[:, 0:128]
    o_ref[1] = h[:, 128:256]


def _tc_linbn(x, W, b, g, be):
    n = x.shape[0]
    return pl.pallas_call(
        _linbn_body,
        out_shape=jax.ShapeDtypeStruct((2, n, D), jnp.float32),
    )(x, W, b[None, :], g[None, :], be[None, :])


def _diva_body(p_ref, cnt_ref, o_ref):
    d = jnp.maximum(cnt_ref[1, 0:M, 0:1], 1.0)
    o_ref[0] = p_ref[0, 0:M, :] / d
    o_ref[1] = p_ref[1, 0:M, :] / d


def _tc_diva(p, cnt):
    return pl.pallas_call(
        _diva_body,
        out_shape=jax.ShapeDtypeStruct((2, M, D), jnp.float32),
    )(p, cnt)


def _divc_body(p_ref, cnt_ref, o_ref):
    d = jnp.maximum(cnt_ref[1, 0:M, 0:1], 1.0)
    o_ref[...] = (p_ref[0, 0:M, :] + p_ref[1, 0:M, :]) / d


def _tc_divc(p, cnt):
    return pl.pallas_call(
        _divc_body,
        out_shape=jax.ShapeDtypeStruct((M, OUT), jnp.float32),
    )(p, cnt)


def _lin2bn_body(v_ref, cnt_ref, w_ref, b_ref, g_ref, be_ref, o_ref):
    d = jnp.maximum(cnt_ref[0, 0:N, 0:1], 1.0)
    h0 = jax.nn.relu(v_ref[0, 0:N, :] / d)
    h1 = jax.nn.relu(v_ref[1, 0:N, :] / d)
    h = (jnp.dot(h0, w_ref[0:128, :], preferred_element_type=jnp.float32)
         + jnp.dot(h1, w_ref[128:256, :], preferred_element_type=jnp.float32))
    h = h + b_ref[...]
    mu = jnp.mean(h, axis=0, keepdims=True)
    var = jnp.mean((h - mu) ** 2, axis=0, keepdims=True)
    h = (h - mu) * lax.rsqrt(var + EPS) * g_ref[...] + be_ref[...]
    o_ref[...] = h


def _tc_lin2bn(v, cnt, W2, b2, g2, be2):
    return pl.pallas_call(
        _lin2bn_body,
        out_shape=jax.ShapeDtypeStruct((N, OUT), jnp.float32),
    )(v, cnt, W2, b2[None, :], g2[None, :], be2[None, :])


BR = 2000


def _heads_body(zs_ref, zf_ref, cs_ref, cf_ref, dW1, db1, dW2, db2, pgW1,
                pgb1, pgW2, pgb2, psW1, psb1, psW2, psb2, z_ref, zsp_ref,
                zfp_ref, xh_ref):
    ds_ = jnp.maximum(cs_ref[0], 1.0)
    df_ = jnp.maximum(cf_ref[0], 1.0)
    zs = (zs_ref[0] + zs_ref[1]) / ds_
    zf = (zf_ref[0] + zf_ref[1]) / df_
    z = zs + zf
    z_ref[...] = z
    h = jax.nn.relu(jnp.dot(z, dW1[...], preferred_element_type=jnp.float32) + db1[...])
    xh_ref[...] = jnp.dot(h, dW2[...], preferred_element_type=jnp.float32) + db2[...]
    hs = jax.nn.relu(jnp.dot(zs, psW1[...], preferred_element_type=jnp.float32) + psb1[...])
    zsp_ref[...] = jnp.dot(hs, psW2[...], preferred_element_type=jnp.float32) + psb2[...]
    hf = jax.nn.relu(jnp.dot(zf, pgW1[...], preferred_element_type=jnp.float32) + pgb1[...])
    zfp_ref[...] = jnp.dot(hf, pgW2[...], preferred_element_type=jnp.float32) + pgb2[...]


def _tc_heads(zs_parts, zf_parts, cnts_s, cnts_f, d_W1, d_b1, d_W2, d_b2,
              pg_W1, pg_b1, pg_W2, pg_b2, ps_W1, ps_b1, ps_W2, ps_b2):
    part = pl.BlockSpec((2, BR, OUT), lambda i: (0, i, 0))
    cnt = pl.BlockSpec((2, BR, 1), lambda i: (0, i, 0))

    def full(a):
        return pl.BlockSpec(a.shape, lambda i: (0,) * a.ndim)

    weights = (d_W1, d_b1[None, :], d_W2, d_b2[None, :], pg_W1,
               pg_b1[None, :], pg_W2, pg_b2[None, :], ps_W1, ps_b1[None, :],
               ps_W2, ps_b2[None, :])
    return pl.pallas_call(
        _heads_body,
        grid=(N // BR,),
        in_specs=[part, part, cnt, cnt] + [full(w) for w in weights],
        out_specs=(
            pl.BlockSpec((BR, OUT), lambda i: (i, 0)),
            pl.BlockSpec((BR, PROJ), lambda i: (i, 0)),
            pl.BlockSpec((BR, PROJ), lambda i: (i, 0)),
            pl.BlockSpec((BR, IN_DIM), lambda i: (i, 0)),
        ),
        out_shape=(
            jax.ShapeDtypeStruct((N, OUT), jnp.float32),
            jax.ShapeDtypeStruct((N, PROJ), jnp.float32),
            jax.ShapeDtypeStruct((N, PROJ), jnp.float32),
            jax.ShapeDtypeStruct((N, IN_DIM), jnp.float32),
        ),
    )(zs_parts, zf_parts, cnts_s, cnts_f, *weights)


_MESH = plsc.VectorSubcoreMesh(core_axis_name="c", subcore_axis_name="s")


def _segsum_loop(plane, gidx_v, sidx_v, acc, nb,
                 bufs, gsems, ssems):
    w = len(bufs)

    def step(i, carry):
        gd = [pltpu.async_copy(plane.at[gidx_v.at[w * i + k]], bufs[k],
                               gsems[k]) for k in range(w)]
        sd = []
        for k in range(w):
            gd[k].wait()
            sd.append(pltpu.async_copy(bufs[k],
                                       acc.at[sidx_v.at[w * i + k]],
                                       ssems[k], add=True))
        for k in range(w):
            sd[k].wait()
        return carry
    lax.fori_loop(0, nb // w, step, 0)


@functools.lru_cache(maxsize=None)
def _sc_pass1():
    nw = 4
    scratch = (
        [pltpu.VMEM((NBP1 // 2, BLK1), jnp.int32),
         pltpu.VMEM((NBP1 // 2, BLK1), jnp.int32)]
        + [pltpu.VMEM((BLK1, D), jnp.float32) for _ in range(nw)]
        + [pltpu.VMEM_SHARED((NPAD, D), jnp.float32)]
        + [pltpu.SemaphoreType.DMA for _ in range(2 * nw)]
    )

    def body(t_hbm, gidx_hbm, sidx_hbm, zeros_hbm, out, *rest):
        gidx_v, sidx_v = rest[0], rest[1]
        bufs = rest[2:2 + nw]
        acc = rest[2 + nw]
        gsems = rest[3 + nw:3 + 2 * nw]
        ssems = rest[3 + 2 * nw:3 + 3 * nw]
        c = lax.axis_index("c")
        s = lax.axis_index("s")
        r0 = s * RPT
        pltpu.sync_copy(zeros_hbm.at[pl.ds(r0, RPT)], acc.at[pl.ds(r0, RPT)])
        plsc.subcore_barrier()
        for hh in range(2):
            h0 = hh * (NBP1 // 2)
            pltpu.sync_copy(gidx_hbm.at[s, pl.ds(h0, NBP1 // 2)], gidx_v)
            pltpu.sync_copy(sidx_hbm.at[s, pl.ds(h0, NBP1 // 2)], sidx_v)
            _segsum_loop(t_hbm.at[c], gidx_v, sidx_v, acc, NBP1 // 2,
                         bufs, gsems, ssems)
        plsc.subcore_barrier()
        pltpu.sync_copy(acc.at[pl.ds(r0, RPT)], out.at[c, pl.ds(r0, RPT)])

    return pl.kernel(
        body,
        out_type=jax.ShapeDtypeStruct((2, NPAD, D), jnp.float32),
        mesh=_MESH, scratch_types=tuple(scratch),
        compiler_params=pltpu.CompilerParams(use_tc_tiling_on_sc=False))


@functools.lru_cache(maxsize=None)
def _sc_pass2():
    scratch = (
        pltpu.VMEM((NB2, BLK), jnp.int32),
        pltpu.VMEM((NB2, BLK), jnp.int32),
        pltpu.VMEM((BLK, OUT), jnp.float32),
        pltpu.VMEM((BLK, OUT), jnp.float32),
        pltpu.VMEM_SHARED((NPAD, OUT), jnp.float32),
        pltpu.SemaphoreType.DMA,
        pltpu.SemaphoreType.DMA,
        pltpu.SemaphoreType.DMA,
        pltpu.SemaphoreType.DMA,
    )

    def body(t_hbm, gidx_hbm, sidx_hbm, zeros_hbm, out,
             gidx_v, sidx_v, buf0, buf1, acc, gs0, gs1, ss0, ss1):
        c = lax.axis_index("c")
        s = lax.axis_index("s")
        wid = c * NS + s
        r0 = s * RPT
        pltpu.sync_copy(gidx_hbm.at[wid], gidx_v)
        pltpu.sync_copy(sidx_hbm.at[wid], sidx_v)
        pltpu.sync_copy(zeros_hbm.at[pl.ds(r0, RPT)], acc.at[pl.ds(r0, RPT)])
        plsc.subcore_barrier()
        _segsum_loop(t_hbm, gidx_v, sidx_v, acc, NB2,
                     (buf0, buf1), (gs0, gs1), (ss0, ss1))
        plsc.subcore_barrier()
        pltpu.sync_copy(acc.at[pl.ds(r0, RPT)], out.at[c, pl.ds(r0, RPT)])

    return pl.kernel(
        body,
        out_type=jax.ShapeDtypeStruct((2, NPAD, OUT), jnp.float32),
        mesh=_MESH, scratch_types=scratch,
        compiler_params=pltpu.CompilerParams(use_tc_tiling_on_sc=False))


HW = 16


@functools.lru_cache(maxsize=None)
def _sc_hist():
    scratch = (
        pltpu.VMEM((NB1, BLK), jnp.int32),
        pltpu.VMEM((128, HW), jnp.float32),
        pltpu.VMEM_SHARED((NPAD, HW), jnp.float32),
        pltpu.SemaphoreType.DMA,
        pltpu.SemaphoreType.DMA,
    )

    def body(idx_hbm, zeros_hbm, ones_hbm, out, idx_v, ones_v, acc, ss0, ss1):
        c = lax.axis_index("c")
        s = lax.axis_index("s")
        r0 = s * RPT
        pltpu.sync_copy(idx_hbm.at[c, s], idx_v)
        pltpu.sync_copy(ones_hbm, ones_v)
        pltpu.sync_copy(zeros_hbm.at[pl.ds(r0, RPT)], acc.at[pl.ds(r0, RPT)])
        plsc.subcore_barrier()

        src = ones_v.at[pl.ds(0, BLK)]

        def step(i, carry):
            s0 = pltpu.async_copy(src, acc.at[idx_v.at[2 * i]], ss0, add=True)
            s1 = pltpu.async_copy(src, acc.at[idx_v.at[2 * i + 1]], ss1,
                                  add=True)
            s0.wait()
            s1.wait()
            return carry
        lax.fori_loop(0, NB1 // 2, step, 0)

        plsc.subcore_barrier()
        pltpu.sync_copy(acc.at[pl.ds(r0, RPT)], out.at[c, pl.ds(r0, RPT)])

    return pl.kernel(
        body,
        out_type=jax.ShapeDtypeStruct((2, NPAD, HW), jnp.float32),
        mesh=_MESH, scratch_types=scratch,
        compiler_params=pltpu.CompilerParams(use_tc_tiling_on_sc=False))


def kernel(x, shg, fhg, s_W1, s_b1, s_g1, s_be1, s_W2, s_b2, s_g2, s_be2,
           f_W1, f_b1, f_g1, f_be1, f_W2, f_b2, f_g2, f_be2,
           d_W1, d_b1, d_W2, d_b2, pg_W1, pg_b1, pg_W2, pg_b2,
           ps_W1, ps_b1, ps_W2, ps_b2):
    zeros = jnp.zeros((NPAD, D), jnp.float32)
    zeros_o = jnp.zeros((NPAD, OUT), jnp.float32)
    ones = jnp.ones((128, HW), jnp.float32)
    zeros_h = jnp.zeros((NPAD, HW), jnp.float32)
    idx = {}
    for g, inc in (("s", shg), ("f", fhg)):
        idx[g] = (inc[0].reshape(NS, NBP1, BLK1),
                  inc[1].reshape(NS, NBP1, BLK1),
                  inc[0].reshape(NC * NS, NB2, BLK),
                  inc[1].reshape(NC * NS, NB2, BLK),
                  inc.reshape(2, NS, NB1, BLK))
    cnts = {g: _sc_hist()(idx[g][4], zeros_h, ones)[:, :, 0:1] for g in "sf"}
    h = {"s": _tc_linbn(x, s_W1, s_b1, s_g1, s_be1),
         "f": _tc_linbn(x, f_W1, f_b1, f_g1, f_be1)}
    e_sums = {g: _sc_pass1()(h[g], idx[g][0], idx[g][1], zeros) for g in "sf"}
    e_feat = {g: _tc_diva(e_sums[g], cnts[g]) for g in "sf"}
    v_sums = {g: _sc_pass1()(e_feat[g], idx[g][1], idx[g][0], zeros)
              for g in "sf"}
    h2 = {"s": _tc_lin2bn(v_sums["s"], cnts["s"], s_W2, s_b2, s_g2, s_be2),
          "f": _tc_lin2bn(v_sums["f"], cnts["f"], f_W2, f_b2, f_g2, f_be2)}
    e2_parts = {g: _sc_pass2()(h2[g], idx[g][2], idx[g][3], zeros_o)
                for g in "sf"}
    e2_feat = {g: _tc_divc(e2_parts[g], cnts[g]) for g in "sf"}
    z_parts = {g: _sc_pass2()(e2_feat[g], idx[g][3], idx[g][2], zeros_o)
               for g in "sf"}
    return _tc_heads(z_parts["s"], z_parts["f"], cnts["s"], cnts["f"],
                     d_W1, d_b1, d_W2, d_b2, pg_W1, pg_b1, pg_W2, pg_b2,
                     ps_W1, ps_b1, ps_W2, ps_b2)


# speedup vs baseline: 1.0517x; 1.0517x over previous
import functools

import jax
import jax.numpy as jnp
from jax import lax
from jax.experimental import pallas as pl
from jax.experimental.pallas import tpu as pltpu
from jax.experimental.pallas import tpu_sc as plsc

N = 10000
M = 10000
NNZ = 160000
IN_DIM = 256
HID = 256
OUT = 64
PROJ = 64
EPS = 1e-5

NC = 2
NS = 16
BLK = 125
NB1 = NNZ // NS // BLK
NB2 = NNZ // (NC * NS) // BLK
BLK1 = 100
NBP1 = NNZ // NS // BLK1
D = 128
NPAD = 10112
RPT = NPAD // NS


def _linbn_body(x_ref, w_ref, b_ref, g_ref, be_ref, o_ref):
    h = jnp.dot(x_ref[...], w_ref[...], preferred_element_type=jnp.float32)
    h = h + b_ref[...]
    mu = jnp.mean(h, axis=0, keepdims=True)
    var = jnp.mean((h - mu) ** 2, axis=0, keepdims=True)
    h = (h - mu) * lax.rsqrt(var + EPS) * g_ref[...] + be_ref[...]
    o_ref[0] = h[:, 0:128]
    o_ref[1] = h[:, 128:256]


def _tc_linbn(x, W, b, g, be):
    n = x.shape[0]
    return pl.pallas_call(
        _linbn_body,
        out_shape=jax.ShapeDtypeStruct((2, n, D), jnp.float32),
    )(x, W, b[None, :], g[None, :], be[None, :])


def _diva_body(p_ref, cnt_ref, o_ref):
    d = jnp.maximum(cnt_ref[1, 0:M, 0:1], 1.0)
    o_ref[0] = p_ref[0, 0:M, :] / d
    o_ref[1] = p_ref[1, 0:M, :] / d


def _tc_diva(p, cnt):
    return pl.pallas_call(
        _diva_body,
        out_shape=jax.ShapeDtypeStruct((2, M, D), jnp.float32),
    )(p, cnt)


def _divc_body(p_ref, cnt_ref, o_ref):
    d = jnp.maximum(cnt_ref[1, 0:M, 0:1], 1.0)
    o_ref[...] = (p_ref[0, 0:M, :] + p_ref[1, 0:M, :]) / d


def _tc_divc(p, cnt):
    return pl.pallas_call(
        _divc_body,
        out_shape=jax.ShapeDtypeStruct((M, OUT), jnp.float32),
    )(p, cnt)


def _lin2bn_body(v_ref, cnt_ref, w_ref, b_ref, g_ref, be_ref, o_ref):
    d = jnp.maximum(cnt_ref[0, 0:N, 0:1], 1.0)
    h0 = jax.nn.relu(v_ref[0, 0:N, :] / d)
    h1 = jax.nn.relu(v_ref[1, 0:N, :] / d)
    h = (jnp.dot(h0, w_ref[0:128, :], preferred_element_type=jnp.float32)
         + jnp.dot(h1, w_ref[128:256, :], preferred_element_type=jnp.float32))
    h = h + b_ref[...]
    mu = jnp.mean(h, axis=0, keepdims=True)
    var = jnp.mean((h - mu) ** 2, axis=0, keepdims=True)
    h = (h - mu) * lax.rsqrt(var + EPS) * g_ref[...] + be_ref[...]
    o_ref[...] = h


def _tc_lin2bn(v, cnt, W2, b2, g2, be2):
    return pl.pallas_call(
        _lin2bn_body,
        out_shape=jax.ShapeDtypeStruct((N, OUT), jnp.float32),
    )(v, cnt, W2, b2[None, :], g2[None, :], be2[None, :])


BR = 2000


def _heads_body(zs_ref, zf_ref, cs_ref, cf_ref, dW1, db1, dW2, db2, pgW1,
                pgb1, pgW2, pgb2, psW1, psb1, psW2, psb2, z_ref, zsp_ref,
                zfp_ref, xh_ref):
    ds_ = jnp.maximum(cs_ref[0], 1.0)
    df_ = jnp.maximum(cf_ref[0], 1.0)
    zs = (zs_ref[0] + zs_ref[1]) / ds_
    zf = (zf_ref[0] + zf_ref[1]) / df_
    z = zs + zf
    z_ref[...] = z
    h = jax.nn.relu(jnp.dot(z, dW1[...], preferred_element_type=jnp.float32) + db1[...])
    xh_ref[...] = jnp.dot(h, dW2[...], preferred_element_type=jnp.float32) + db2[...]
    hs = jax.nn.relu(jnp.dot(zs, psW1[...], preferred_element_type=jnp.float32) + psb1[...])
    zsp_ref[...] = jnp.dot(hs, psW2[...], preferred_element_type=jnp.float32) + psb2[...]
    hf = jax.nn.relu(jnp.dot(zf, pgW1[...], preferred_element_type=jnp.float32) + pgb1[...])
    zfp_ref[...] = jnp.dot(hf, pgW2[...], preferred_element_type=jnp.float32) + pgb2[...]


def _tc_heads(zs_parts, zf_parts, cnts_s, cnts_f, d_W1, d_b1, d_W2, d_b2,
              pg_W1, pg_b1, pg_W2, pg_b2, ps_W1, ps_b1, ps_W2, ps_b2):
    part = pl.BlockSpec((2, BR, OUT), lambda i: (0, i, 0))
    cnt = pl.BlockSpec((2, BR, 1), lambda i: (0, i, 0))

    def full(a):
        return pl.BlockSpec(a.shape, lambda i: (0,) * a.ndim)

    weights = (d_W1, d_b1[None, :], d_W2, d_b2[None, :], pg_W1,
               pg_b1[None, :], pg_W2, pg_b2[None, :], ps_W1, ps_b1[None, :],
               ps_W2, ps_b2[None, :])
    return pl.pallas_call(
        _heads_body,
        grid=(N // BR,),
        in_specs=[part, part, cnt, cnt] + [full(w) for w in weights],
        out_specs=(
            pl.BlockSpec((BR, OUT), lambda i: (i, 0)),
            pl.BlockSpec((BR, PROJ), lambda i: (i, 0)),
            pl.BlockSpec((BR, PROJ), lambda i: (i, 0)),
            pl.BlockSpec((BR, IN_DIM), lambda i: (i, 0)),
        ),
        out_shape=(
            jax.ShapeDtypeStruct((N, OUT), jnp.float32),
            jax.ShapeDtypeStruct((N, PROJ), jnp.float32),
            jax.ShapeDtypeStruct((N, PROJ), jnp.float32),
            jax.ShapeDtypeStruct((N, IN_DIM), jnp.float32),
        ),
    )(zs_parts, zf_parts, cnts_s, cnts_f, *weights)


_MESH = plsc.VectorSubcoreMesh(core_axis_name="c", subcore_axis_name="s")


def _segsum_loop(plane, gidx_v, sidx_v, acc, nb,
                 bufs, gsems, ssems):
    w = len(bufs)

    def step(i, carry):
        gd = [pltpu.async_copy(plane.at[gidx_v.at[w * i + k]], bufs[k],
                               gsems[k]) for k in range(w)]
        sd = []
        for k in range(w):
            gd[k].wait()
            sd.append(pltpu.async_copy(bufs[k],
                                       acc.at[sidx_v.at[w * i + k]],
                                       ssems[k], add=True))
        for k in range(w):
            sd[k].wait()
        return carry
    lax.fori_loop(0, nb // w, step, 0)


@functools.lru_cache(maxsize=None)
def _sc_pass1():
    nw = 3
    scratch = (
        [pltpu.VMEM((NBP1 // 2, BLK1), jnp.int32),
         pltpu.VMEM((NBP1 // 2, BLK1), jnp.int32)]
        + [pltpu.VMEM((BLK1, D), jnp.float32) for _ in range(nw)]
        + [pltpu.VMEM_SHARED((NPAD, D), jnp.float32)]
        + [pltpu.SemaphoreType.DMA for _ in range(2 * nw)]
    )

    def body(t_hbm, gidx_hbm, sidx_hbm, zeros_hbm, out, *rest):
        gidx_v, sidx_v = rest[0], rest[1]
        bufs = rest[2:2 + nw]
        acc = rest[2 + nw]
        gsems = rest[3 + nw:3 + 2 * nw]
        ssems = rest[3 + 2 * nw:3 + 3 * nw]
        c = lax.axis_index("c")
        s = lax.axis_index("s")
        r0 = s * RPT
        pltpu.sync_copy(zeros_hbm.at[pl.ds(r0, RPT)], acc.at[pl.ds(r0, RPT)])
        plsc.subcore_barrier()
        for hh in range(2):
            h0 = hh * (NBP1 // 2)
            pltpu.sync_copy(gidx_hbm.at[s, pl.ds(h0, NBP1 // 2)], gidx_v)
            pltpu.sync_copy(sidx_hbm.at[s, pl.ds(h0, NBP1 // 2)], sidx_v)
            _segsum_loop(t_hbm.at[c], gidx_v, sidx_v, acc, NBP1 // 2,
                         bufs, gsems, ssems)
        plsc.subcore_barrier()
        pltpu.sync_copy(acc.at[pl.ds(r0, RPT)], out.at[c, pl.ds(r0, RPT)])

    return pl.kernel(
        body,
        out_type=jax.ShapeDtypeStruct((2, NPAD, D), jnp.float32),
        mesh=_MESH, scratch_types=tuple(scratch),
        compiler_params=pltpu.CompilerParams(use_tc_tiling_on_sc=False))


@functools.lru_cache(maxsize=None)
def _sc_pass2():
    scratch = (
        pltpu.VMEM((NB2, BLK), jnp.int32),
        pltpu.VMEM((NB2, BLK), jnp.int32),
        pltpu.VMEM((BLK, OUT), jnp.float32),
        pltpu.VMEM((BLK, OUT), jnp.float32),
        pltpu.VMEM_SHARED((NPAD, OUT), jnp.float32),
        pltpu.SemaphoreType.DMA,
        pltpu.SemaphoreType.DMA,
        pltpu.SemaphoreType.DMA,
        pltpu.SemaphoreType.DMA,
    )

    def body(t_hbm, gidx_hbm, sidx_hbm, zeros_hbm, out,
             gidx_v, sidx_v, buf0, buf1, acc, gs0, gs1, ss0, ss1):
        c = lax.axis_index("c")
        s = lax.axis_index("s")
        wid = c * NS + s
        r0 = s * RPT
        pltpu.sync_copy(gidx_hbm.at[wid], gidx_v)
        pltpu.sync_copy(sidx_hbm.at[wid], sidx_v)
        pltpu.sync_copy(zeros_hbm.at[pl.ds(r0, RPT)], acc.at[pl.ds(r0, RPT)])
        plsc.subcore_barrier()
        _segsum_loop(t_hbm, gidx_v, sidx_v, acc, NB2,
                     (buf0, buf1), (gs0, gs1), (ss0, ss1))
        plsc.subcore_barrier()
        pltpu.sync_copy(acc.at[pl.ds(r0, RPT)], out.at[c, pl.ds(r0, RPT)])

    return pl.kernel(
        body,
        out_type=jax.ShapeDtypeStruct((2, NPAD, OUT), jnp.float32),
        mesh=_MESH, scratch_types=scratch,
        compiler_params=pltpu.CompilerParams(use_tc_tiling_on_sc=False))


HW = 16


@functools.lru_cache(maxsize=None)
def _sc_hist():
    scratch = (
        pltpu.VMEM((NB1, BLK), jnp.int32),
        pltpu.VMEM((128, HW), jnp.float32),
        pltpu.VMEM_SHARED((NPAD, HW), jnp.float32),
        pltpu.SemaphoreType.DMA,
        pltpu.SemaphoreType.DMA,
    )

    def body(idx_hbm, zeros_hbm, ones_hbm, out, idx_v, ones_v, acc, ss0, ss1):
        c = lax.axis_index("c")
        s = lax.axis_index("s")
        r0 = s * RPT
        pltpu.sync_copy(idx_hbm.at[c, s], idx_v)
        pltpu.sync_copy(ones_hbm, ones_v)
        pltpu.sync_copy(zeros_hbm.at[pl.ds(r0, RPT)], acc.at[pl.ds(r0, RPT)])
        plsc.subcore_barrier()

        src = ones_v.at[pl.ds(0, BLK)]

        def step(i, carry):
            s0 = pltpu.async_copy(src, acc.at[idx_v.at[2 * i]], ss0, add=True)
            s1 = pltpu.async_copy(src, acc.at[idx_v.at[2 * i + 1]], ss1,
                                  add=True)
            s0.wait()
            s1.wait()
            return carry
        lax.fori_loop(0, NB1 // 2, step, 0)

        plsc.subcore_barrier()
        pltpu.sync_copy(acc.at[pl.ds(r0, RPT)], out.at[c, pl.ds(r0, RPT)])

    return pl.kernel(
        body,
        out_type=jax.ShapeDtypeStruct((2, NPAD, HW), jnp.float32),
        mesh=_MESH, scratch_types=scratch,
        compiler_params=pltpu.CompilerParams(use_tc_tiling_on_sc=False))


def kernel(x, shg, fhg, s_W1, s_b1, s_g1, s_be1, s_W2, s_b2, s_g2, s_be2,
           f_W1, f_b1, f_g1, f_be1, f_W2, f_b2, f_g2, f_be2,
           d_W1, d_b1, d_W2, d_b2, pg_W1, pg_b1, pg_W2, pg_b2,
           ps_W1, ps_b1, ps_W2, ps_b2):
    zeros = jnp.zeros((NPAD, D), jnp.float32)
    zeros_o = jnp.zeros((NPAD, OUT), jnp.float32)
    ones = jnp.ones((128, HW), jnp.float32)
    zeros_h = jnp.zeros((NPAD, HW), jnp.float32)
    idx = {}
    for g, inc in (("s", shg), ("f", fhg)):
        idx[g] = (inc[0].reshape(NS, NBP1, BLK1),
                  inc[1].reshape(NS, NBP1, BLK1),
                  inc[0].reshape(NC * NS, NB2, BLK),
                  inc[1].reshape(NC * NS, NB2, BLK),
                  inc.reshape(2, NS, NB1, BLK))
    cnts = {g: _sc_hist()(idx[g][4], zeros_h, ones)[:, :, 0:1] for g in "sf"}
    h = {"s": _tc_linbn(x, s_W1, s_b1, s_g1, s_be1),
         "f": _tc_linbn(x, f_W1, f_b1, f_g1, f_be1)}
    e_sums = {g: _sc_pass1()(h[g], idx[g][0], idx[g][1], zeros) for g in "sf"}
    e_feat = {g: _tc_diva(e_sums[g], cnts[g]) for g in "sf"}
    v_sums = {g: _sc_pass1()(e_feat[g], idx[g][1], idx[g][0], zeros)
              for g in "sf"}
    h2 = {"s": _tc_lin2bn(v_sums["s"], cnts["s"], s_W2, s_b2, s_g2, s_be2),
          "f": _tc_lin2bn(v_sums["f"], cnts["f"], f_W2, f_b2, f_g2, f_be2)}
    e2_parts = {g: _sc_pass2()(h2[g], idx[g][2], idx[g][3], zeros_o)
                for g in "sf"}
    e2_feat = {g: _tc_divc(e2_parts[g], cnts[g]) for g in "sf"}
    z_parts = {g: _sc_pass2()(e2_feat[g], idx[g][3], idx[g][2], zeros_o)
               for g in "sf"}
    return _tc_heads(z_parts["s"], z_parts["f"], cnts["s"], cnts["f"],
                     d_W1, d_b1, d_W2, d_b2, pg_W1, pg_b1, pg_W2, pg_b2,
                     ps_W1, ps_b1, ps_W2, ps_b2)
